# Initial kernel scaffold; baseline (speedup 1.0000x reference)
import jax
import jax.numpy as jnp
from jax.experimental import pallas as pl


def kernel(Corpus_, batch_inputs, entity_embeddings, relation_embed, type_embed, edge_list, edge_type, edge_embed, a_h0, a2_h0, went_h0, a_h1, a2_h1, went_h1, a_out, a2_out, went_out, W):
    raise NotImplementedError("write your pallas kernel here")


# speedup vs baseline: 5.3904x; 5.3904x over previous
import functools

import jax
import jax.numpy as jnp
from jax.experimental import pallas as pl
from jax.experimental.pallas import tpu as pltpu

NT = 500
TP = 512
NR = 200
RP = 256
B = 512
NE_ENT = 10000


def _elu(x):
    return jnp.where(x > 0, x, jnp.exp(x) - 1.0)


def _lrelu_exp(s):
    return jnp.exp(-jnp.where(s > 0, s, 0.2 * s))


def _e1_body(e0_ref, e1_ref, eeT_ref, entP_ref, typeP_ref, A1T_ref, A2T_ref,
             ArM_ref, a2R_ref, accV_ref, accS_ref, T2Ts_ref,
             T1Ts_s, s2S_s, ceS_s):
    i = pl.program_id(0)

    @pl.when(i == 0)
    def _init():
        accV_ref[...] = jnp.zeros_like(accV_ref)
        accS_ref[...] = jnp.zeros_like(accS_ref)
        s2rows = []
        cerows = []
        for h in range(2):
            a2row = a2R_ref[h:h + 1, :]
            T1T = jax.lax.dot_general(
                A1T_ref[h], entP_ref[...],
                (((0,), (1,)), ((), ())), preferred_element_type=jnp.float32)
            s1row = jnp.dot(a2row, T1T, preferred_element_type=jnp.float32)
            T1Ts_s[h] = jnp.concatenate(
                [T1T, s1row, jnp.zeros((7, TP), jnp.float32)], axis=0)
            T2T = jax.lax.dot_general(
                A2T_ref[h], typeP_ref[...],
                (((0,), (1,)), ((), ())), preferred_element_type=jnp.float32)
            T2Ts_ref[h] = T2T
            s2rows.append(jnp.dot(a2row, T2T, preferred_element_type=jnp.float32))
            cerows.append(jnp.dot(a2row, ArM_ref[h], preferred_element_type=jnp.float32))
        s2S_s[...] = jnp.concatenate(s2rows + [jnp.zeros((6, TP), jnp.float32)], axis=0)
        ceS_s[...] = jnp.concatenate(cerows + [jnp.zeros((6, 32), jnp.float32)], axis=0)

    e0 = e0_ref[0]
    e1 = e1_ref[0]
    ee = eeT_ref[...]
    iotaT = jax.lax.broadcasted_iota(jnp.int32, (TP, B), 0)
    O0T = (iotaT == e0).astype(jnp.float32)
    O1T = (iotaT == e1).astype(jnp.float32)
    for h in range(2):
        TgT = jnp.dot(T1Ts_s[h], O0T, preferred_element_type=jnp.float32)
        s2g = jnp.dot(s2S_s[h:h + 1, :], O1T, preferred_element_type=jnp.float32)
        se = jnp.dot(ceS_s[h:h + 1, :], ee, preferred_element_type=jnp.float32)
        s = TgT[32:33, :] + s2g + se
        w = _lrelu_exp(s)
        V = jnp.concatenate([TgT[:32, :] * w, ee * w], axis=0)
        accV_ref[h] += jax.lax.dot_general(
            V, O1T, (((1,), (1,)), ((), ())), preferred_element_type=jnp.float32)
        w8 = jnp.concatenate([w, jnp.zeros((7, B), jnp.float32)], axis=0)
        accS_ref[h] += jax.lax.dot_general(
            w8, O1T, (((1,), (1,)), ((), ())), preferred_element_type=jnp.float32)


def _mid_body(ent_ref, wcat_ref, wout_ref, accV_ref, accS_ref, T2Ts_ref,
              ArM_ref, relP_ref, W_ref, A1o_ref, A2o_ref, Aro_ref, a2o_ref,
              out1_ref, TABo_ref, S2o8_ref, T2oT_ref, RTs_ref, orel_ref):
    i = pl.program_id(0)
    h1 = jnp.dot(ent_ref[...], wcat_ref[...], preferred_element_type=jnp.float32)
    x1c = _elu(h1)
    out1_ref[...] = _elu(jnp.dot(x1c, wout_ref[...], preferred_element_type=jnp.float32))

    @pl.when(i == 0)
    def _tables():
        h2Ts = []
        for h in range(2):
            rs = accS_ref[h, 0:1, :]
            rsafe = jnp.where(rs == 0.0, 1e-12, rs)
            numT = accV_ref[h, :32, :] + jnp.dot(
                ArM_ref[h], accV_ref[h, 32:64, :], preferred_element_type=jnp.float32)
            h2T = numT / rsafe + jnp.where(rs > 0.0, T2Ts_ref[h], 0.0)
            h2Ts.append(_elu(h2T))
        x2cT = jnp.concatenate(h2Ts, axis=0)
        orelP = jnp.dot(relP_ref[...], W_ref[...], preferred_element_type=jnp.float32)
        orel_ref[...] = orelP[:NR, :]
        x1c512 = x1c[:TP, :]
        T1oT = jax.lax.dot_general(A1o_ref[...], x1c512,
                                   (((1,), (1,)), ((), ())), preferred_element_type=jnp.float32)
        s1o = jnp.dot(a2o_ref[0:1, :], T1oT, preferred_element_type=jnp.float32)
        TABo_ref[...] = jnp.concatenate(
            [T1oT, s1o, jnp.zeros((7, TP), jnp.float32)], axis=0)
        T2oT = jnp.dot(A2o_ref[...], x2cT, preferred_element_type=jnp.float32)
        T2oT_ref[...] = T2oT
        s2o = jnp.dot(a2o_ref[0:1, :], T2oT, preferred_element_type=jnp.float32)
        S2o8_ref[...] = jnp.concatenate([s2o, jnp.zeros((7, TP), jnp.float32)], axis=0)
        RT = jax.lax.dot_general(Aro_ref[...], orelP,
                                 (((1,), (1,)), ((), ())), preferred_element_type=jnp.float32)
        ser = jnp.dot(a2o_ref[0:1, :], RT, preferred_element_type=jnp.float32)
        RTs_ref[...] = jnp.concatenate(
            [RT, ser, jnp.zeros((7, RP), jnp.float32)], axis=0)


def _e2_body(e0_ref, e1_ref, et_ref, TABo_ref, S2o8_ref, RTs_ref, T2oT_ref,
             accV_ref, accS_ref, x2T_ref, nblk):
    i = pl.program_id(0)

    @pl.when(i == 0)
    def _init():
        accV_ref[...] = jnp.zeros_like(accV_ref)
        accS_ref[...] = jnp.zeros_like(accS_ref)

    e0 = e0_ref[0]
    e1 = e1_ref[0]
    et = et_ref[0]
    iotaT = jax.lax.broadcasted_iota(jnp.int32, (TP, B), 0)
    iotaR = jax.lax.broadcasted_iota(jnp.int32, (RP, B), 0)
    O0T = (iotaT == e0).astype(jnp.float32)
    O1T = (iotaT == e1).astype(jnp.float32)
    OrT = (iotaR == et).astype(jnp.float32)
    TgT = jnp.dot(TABo_ref[...], O0T, preferred_element_type=jnp.float32)
    RgT = jnp.dot(RTs_ref[...], OrT, preferred_element_type=jnp.float32)
    s2g = jnp.dot(S2o8_ref[...], O1T, preferred_element_type=jnp.float32)
    s = TgT[64:65, :] + RgT[64:65, :] + s2g[0:1, :]
    w = _lrelu_exp(s)
    V = (TgT[:64, :] + RgT[:64, :]) * w
    accV_ref[...] += jax.lax.dot_general(
        V, O1T, (((1,), (1,)), ((), ())), preferred_element_type=jnp.float32)
    w8 = jnp.concatenate([w, jnp.zeros((7, B), jnp.float32)], axis=0)
    accS_ref[...] += jax.lax.dot_general(
        w8, O1T, (((1,), (1,)), ((), ())), preferred_element_type=jnp.float32)

    @pl.when(i == nblk - 1)
    def _fin():
        rs = accS_ref[0:1, :]
        rsafe = jnp.where(rs == 0.0, 1e-12, rs)
        x2fT = accV_ref[...] / rsafe + jnp.where(rs > 0.0, T2oT_ref[...], 0.0)
        x2T_ref[...] = _elu(x2fT)


def kernel(Corpus_, batch_inputs, entity_embeddings, relation_embed, type_embed,
           edge_list, edge_type, edge_embed,
           a_h0, a2_h0, went_h0, a_h1, a2_h1, went_h1,
           a_out, a2_out, went_out, W):
    f32 = jnp.float32
    E = edge_list.shape[1]
    nblk = E // B

    e0r = edge_list[0].reshape(nblk, 1, B)
    e1r = edge_list[1].reshape(nblk, 1, B)
    etr = edge_type.reshape(nblk, 1, B)
    eeT = edge_embed.T

    entP = entity_embeddings[:TP]
    typeP = jnp.pad(type_embed, ((0, TP - NT), (0, 0)))
    relP = jnp.pad(relation_embed, ((0, RP - NR), (0, 0)))

    A1T = jnp.stack([a_h0.T[0:64], a_h1.T[0:64]])
    A2T = jnp.stack([a_h0.T[64:128], a_h1.T[64:128]])
    ArT = jnp.stack([a_h0.T[128:160], a_h1.T[128:160]])
    ArM = jnp.stack([a_h0[:, 128:160], a_h1[:, 128:160]])
    a2R = jnp.stack([a2_h0, a2_h1])
    wcat = jnp.concatenate([went_h0, went_h1], axis=1)
    A1o = a_out[:, 0:64]
    A2o = a_out[:, 64:128]
    Aro = a_out[:, 128:192]
    a2oR = jnp.pad(a2_out.reshape(1, 64), ((0, 7), (0, 0)))

    full3 = lambda shp: pl.BlockSpec(shp, lambda i: (0,) * len(shp))
    idx3 = pl.BlockSpec((1, 1, B), lambda i: (i, 0, 0))

    accV, accS, T2Ts = pl.pallas_call(
        _e1_body,
        grid=(nblk,),
        in_specs=[idx3, idx3,
                  pl.BlockSpec((32, B), lambda i: (0, i)),
                  full3((TP, 64)), full3((TP, 64)),
                  full3((2, 64, 32)), full3((2, 64, 32)), full3((2, 32, 32)),
                  full3((2, 32))],
        out_specs=[full3((2, 64, TP)), full3((2, 8, TP)), full3((2, 32, TP))],
        out_shape=[jax.ShapeDtypeStruct((2, 64, TP), f32),
                   jax.ShapeDtypeStruct((2, 8, TP), f32),
                   jax.ShapeDtypeStruct((2, 32, TP), f32)],
        scratch_shapes=[pltpu.VMEM((2, 40, TP), f32),
                        pltpu.VMEM((8, TP), f32),
                        pltpu.VMEM((8, 32), f32)],
    )(e0r, e1r, eeT, entP, typeP, A1T, A2T, ArM, a2R)

    out1, TABo, S2o8, T2oT, RTs, orel = pl.pallas_call(
        _mid_body,
        grid=(10,),
        in_specs=[pl.BlockSpec((1000, 64), lambda i: (i, 0)),
                  full3((64, 64)), full3((64, 64)),
                  full3((2, 64, TP)), full3((2, 8, TP)), full3((2, 32, TP)),
                  full3((2, 32, 32)), full3((RP, 32)), full3((32, 64)),
                  full3((64, 64)), full3((64, 64)), full3((64, 64)),
                  full3((8, 64))],
        out_specs=[pl.BlockSpec((1000, 64), lambda i: (i, 0)),
                   full3((72, TP)), full3((8, TP)), full3((64, TP)),
                   full3((72, RP)), full3((NR, 64))],
        out_shape=[jax.ShapeDtypeStruct((NE_ENT, 64), f32),
                   jax.ShapeDtypeStruct((72, TP), f32),
                   jax.ShapeDtypeStruct((8, TP), f32),
                   jax.ShapeDtypeStruct((64, TP), f32),
                   jax.ShapeDtypeStruct((72, RP), f32),
                   jax.ShapeDtypeStruct((NR, 64), f32)],
    )(entity_embeddings, wcat, went_out, accV, accS, T2Ts,
      ArM, relP, W, A1o, A2o, Aro, a2oR)

    _, _, x2T = pl.pallas_call(
        functools.partial(_e2_body, nblk=nblk),
        grid=(nblk,),
        in_specs=[idx3, idx3, idx3,
                  full3((72, TP)), full3((8, TP)), full3((72, RP)),
                  full3((64, TP))],
        out_specs=[full3((64, TP)), full3((8, TP)), full3((64, TP))],
        out_shape=[jax.ShapeDtypeStruct((64, TP), f32),
                   jax.ShapeDtypeStruct((8, TP), f32),
                   jax.ShapeDtypeStruct((64, TP), f32)],
    )(e0r, e1r, etr, TABo, S2o8, RTs, T2oT)

    out2 = x2T.T[:NT]
    return out1, out2, orel


# speedup vs baseline: 8.8195x; 1.6361x over previous
import functools

import jax
import jax.numpy as jnp
from jax import lax
from jax.experimental import pallas as pl
from jax.experimental.pallas import tpu as pltpu
from jax.experimental.pallas import tpu_sc as plsc

NT = 500
TP = 512
NR = 200
RP = 256
NE_ENT = 10000
PB = 3200
NW = 32
CHUNK = 80


def _elu(x):
    return jnp.where(x > 0, x, jnp.exp(x) - 1.0)


def _lrelu_exp(s):
    return jnp.exp(-jnp.where(s > 0, s, 0.2 * s))


def _take(x, idx):
    return x.at[idx].get(mode="promise_in_bounds", unique_indices=False)


def _prep_body(ee_ref, entP_ref, typeP_ref, A1cat_ref, A2cat_ref, ArT_ref,
               a2R_ref, pes_ref, T1cat_ref, S1_ref, S2_ref, T2cat_ref, W80_s):
    i = pl.program_id(0)

    @pl.when(i == 0)
    def _tables():
        T1cat = jnp.dot(entP_ref[...], A1cat_ref[...], preferred_element_type=jnp.float32)
        T2cat = jnp.dot(typeP_ref[...], A2cat_ref[...], preferred_element_type=jnp.float32)
        T1cat_ref[...] = T1cat
        T2cat_ref[...] = T2cat
        s1r, s2r, cecols = [], [], []
        for h in range(2):
            a2row = a2R_ref[h:h + 1, :]
            s1r.append(jax.lax.dot_general(
                a2row, T1cat[:, 32 * h:32 * h + 32],
                (((1,), (1,)), ((), ())), preferred_element_type=jnp.float32))
            s2r.append(jax.lax.dot_general(
                a2row, T2cat[:, 32 * h:32 * h + 32],
                (((1,), (1,)), ((), ())), preferred_element_type=jnp.float32))
            cecols.append(jax.lax.dot_general(
                ArT_ref[h], a2row,
                (((1,), (1,)), ((), ())), preferred_element_type=jnp.float32))
        S1_ref[...] = jnp.concatenate(s1r + [jnp.zeros((6, TP), jnp.float32)], axis=0)
        S2_ref[...] = jnp.concatenate(s2r + [jnp.zeros((6, TP), jnp.float32)], axis=0)
        w128 = jnp.concatenate(
            [ArT_ref[0], ArT_ref[1], cecols[0], cecols[1],
             jnp.zeros((32, 62), jnp.float32)], axis=1)
        z = jnp.zeros((32, 128), jnp.float32)
        W80_s[...] = jnp.concatenate([
            jnp.concatenate([w128, z, z, z], axis=1),
            jnp.concatenate([z, w128, z, z], axis=1),
            jnp.concatenate([z, z, w128, z], axis=1),
            jnp.concatenate([z, z, z, w128], axis=1),
        ], axis=0)

    pes_ref[...] = jnp.dot(ee_ref[...], W80_s[...], preferred_element_type=jnp.float32)


def _e1sc_body(pes_hbm, el0_hbm, el1_hbm, t1_hbm, s1_hbm, s2_hbm, out_hbm,
               t1_v, s1a, s1b, s2a, s2b, acc0, acc1, pes_a, pes_b,
               e0_v, e1_v, sem0, sem1, nch):
    i32 = jnp.int32
    wid = lax.axis_index("s") * 2 + lax.axis_index("c")
    ebase = wid * (nch * CHUNK)
    lane = lax.iota(i32, 16)
    lane0f = jnp.where(lane == 0, 1.0, 0.0).astype(jnp.float32)

    ew = nch * CHUNK
    pltpu.make_async_copy(el0_hbm.at[pl.ds(ebase, ew)], e0_v, sem0).start()
    pltpu.make_async_copy(el1_hbm.at[pl.ds(ebase, ew)], e1_v, sem0).start()
    pltpu.sync_copy(t1_hbm, t1_v)
    pltpu.sync_copy(s1_hbm.at[pl.ds(0, TP)], s1a)
    pltpu.sync_copy(s1_hbm.at[pl.ds(TP, TP)], s1b)
    pltpu.sync_copy(s2_hbm.at[pl.ds(0, TP)], s2a)
    pltpu.sync_copy(s2_hbm.at[pl.ds(TP, TP)], s2b)

    def _zero(r):
        acc0[pl.ds(r * 16, 16)] = jnp.zeros((16,), jnp.float32)
        acc1[pl.ds(r * 16, 16)] = jnp.zeros((16,), jnp.float32)
    plsc.parallel_loop(0, 3 * TP, unroll=4)(_zero)
    pltpu.make_async_copy(el0_hbm.at[pl.ds(ebase, ew)], e0_v, sem0).wait()
    pltpu.make_async_copy(el1_hbm.at[pl.ds(ebase, ew)], e1_v, sem0).wait()

    bufs = ((pes_a, sem0), (pes_b, sem1))

    def _copies(ci, b):
        base = ebase + ci * CHUNK
        pes_v, sem = bufs[b]
        return (
            pltpu.make_async_copy(pes_hbm.at[pl.ds(base * 128, CHUNK * 128)], pes_v, sem),
        )

    def _fire(ci, b):
        for c in _copies(ci, b):
            c.start()

    def _wait(ci, b):
        for c in _copies(ci, b):
            c.wait()

    def _process(ci, b):
        pes_v, _ = bufs[b]
        coff = ci * CHUNK

        def _group(g, _):
            off = g * 16
            e0i = e0_v[pl.ds(coff + off, 16)]
            e1i = e1_v[pl.ds(coff + off, 16)]
            s1g0 = plsc.load_gather(s1a, [e0i])
            s1g1 = plsc.load_gather(s1b, [e0i])
            s2g0 = plsc.load_gather(s2a, [e1i])
            s2g1 = plsc.load_gather(s2b, [e1i])
            pbase = (lane + off) * 128
            se0 = plsc.load_gather(pes_v, [pbase + 64])
            se1 = plsc.load_gather(pes_v, [pbase + 65])
            w0 = _lrelu_exp(s1g0 + s2g0 + se0)
            w1 = _lrelu_exp(s1g1 + s2g1 + se1)
            e0m = e0i * 64
            e1m = e1i * 48
            for j in range(16):
                fj = jnp.full((16,), j, i32)
                tB = _take(e0m, fj) + lane
                aB = _take(e1m, fj) + lane
                w0b = _take(w0, fj)
                w1b = _take(w1, fj)
                pr = (off + j) * 128
                for h, (wb, acc) in enumerate(((w0b, acc0), (w1b, acc1))):
                    t0 = plsc.load_gather(t1_v, [tB + 32 * h])
                    t1r = plsc.load_gather(t1_v, [tB + (32 * h + 16)])
                    p0 = pes_v[pl.ds(pr + 32 * h, 16)]
                    p1 = pes_v[pl.ds(pr + 32 * h + 16, 16)]
                    plsc.addupdate_scatter(acc, [aB], wb * (t0 + p0))
                    plsc.addupdate_scatter(acc, [aB + 16], wb * (t1r + p1))
                    plsc.addupdate_scatter(acc, [aB + 32], wb * lane0f)
            return 0
        plsc.parallel_loop(0, CHUNK // 16, unroll=1)(
            lambda g: (_group(g, 0), None)[1])

    _fire(0, 0)

    def _pair(p, _):
        ci0 = 2 * p
        _fire(ci0 + 1, 1)
        _wait(ci0, 0)
        _process(ci0, 0)
        _fire(ci0 + 2, 0)
        _wait(ci0 + 1, 1)
        _process(ci0 + 1, 1)
        return 0
    lax.fori_loop(0, (nch - 1) // 2, _pair, 0)
    _wait(nch - 1, 0)
    _process(nch - 1, 0)

    pltpu.sync_copy(acc0, out_hbm.at[pl.ds((wid * 2) * (3 * TP * 16), 3 * TP * 16)])
    pltpu.sync_copy(acc1, out_hbm.at[pl.ds((wid * 2 + 1) * (3 * TP * 16), 3 * TP * 16)])


def _mid_body(ent_ref, wcat_ref, wout_ref, accE1_ref, T2cat_ref, relP_ref,
              W_ref, A1o_ref, A2o_ref, Aro_ref, a2o_ref,
              out1_ref, T1o_ref, SCo_ref, Rtab_ref, SEr_ref, T2o_ref, orel_ref):
    i = pl.program_id(0)
    h1 = jnp.dot(ent_ref[...], wcat_ref[...], preferred_element_type=jnp.float32)
    x1c = _elu(h1)
    out1_ref[...] = _elu(jnp.dot(x1c, wout_ref[...], preferred_element_type=jnp.float32))

    @pl.when(i == 0)
    def _tables():
        acc = jnp.sum(accE1_ref[...], axis=0)
        h2s = []
        for h in range(2):
            numer = acc[h, :, 0:32]
            rs = acc[h, :, 32:33]
            rsafe = jnp.where(rs == 0.0, 1e-12, rs)
            h2 = numer / rsafe + jnp.where(
                rs > 0.0, T2cat_ref[:, 32 * h:32 * h + 32], 0.0)
            h2s.append(_elu(h2))
        x2c = jnp.concatenate(h2s, axis=1)
        a2o = a2o_ref[0:1, :]
        x1c512 = x1c[:TP, :]
        T1o = jax.lax.dot_general(x1c512, A1o_ref[...],
                                  (((1,), (1,)), ((), ())), preferred_element_type=jnp.float32)
        T1o_ref[...] = T1o
        s1o = jax.lax.dot_general(a2o, T1o, (((1,), (1,)), ((), ())),
                                  preferred_element_type=jnp.float32)
        T2o = jax.lax.dot_general(x2c, A2o_ref[...],
                                  (((1,), (1,)), ((), ())), preferred_element_type=jnp.float32)
        T2o_ref[...] = T2o
        s2o = jax.lax.dot_general(a2o, T2o, (((1,), (1,)), ((), ())),
                                  preferred_element_type=jnp.float32)
        SCo_ref[...] = jnp.concatenate(
            [s1o, s2o, jnp.zeros((6, TP), jnp.float32)], axis=0)
        orelP = jnp.dot(relP_ref[...], W_ref[...], preferred_element_type=jnp.float32)
        orel_ref[...] = orelP[:NR, :]
        Rtab = jax.lax.dot_general(orelP, Aro_ref[...],
                                   (((1,), (1,)), ((), ())), preferred_element_type=jnp.float32)
        Rtab_ref[...] = Rtab
        ser = jax.lax.dot_general(a2o, Rtab, (((1,), (1,)), ((), ())),
                                  preferred_element_type=jnp.float32)
        SEr_ref[...] = jnp.concatenate(
            [ser, jnp.zeros((7, RP), jnp.float32)], axis=0)


def _e2sc_body(el0_hbm, el1_hbm, et_hbm, t1o_hbm, r_hbm, sco_hbm, ser_hbm,
               out_hbm, t1o_v, r_v, s1o_v, s2o_v, ser_v, acc,
               e0_v, e1_v, et_v, sem0, nch):
    i32 = jnp.int32
    wid = lax.axis_index("s") * 2 + lax.axis_index("c")
    ebase = wid * (nch * CHUNK)
    ew = nch * CHUNK
    lane = lax.iota(i32, 16)
    lane0f = jnp.where(lane == 0, 1.0, 0.0).astype(jnp.float32)

    pltpu.make_async_copy(el0_hbm.at[pl.ds(ebase, ew)], e0_v, sem0).start()
    pltpu.make_async_copy(el1_hbm.at[pl.ds(ebase, ew)], e1_v, sem0).start()
    pltpu.make_async_copy(et_hbm.at[pl.ds(ebase, ew)], et_v, sem0).start()
    pltpu.sync_copy(t1o_hbm, t1o_v)
    pltpu.sync_copy(r_hbm, r_v)
    pltpu.sync_copy(sco_hbm.at[pl.ds(0, TP)], s1o_v)
    pltpu.sync_copy(sco_hbm.at[pl.ds(TP, TP)], s2o_v)
    pltpu.sync_copy(ser_hbm.at[pl.ds(0, RP)], ser_v)

    def _zero(r):
        acc[pl.ds(r * 16, 16)] = jnp.zeros((16,), jnp.float32)
    plsc.parallel_loop(0, 5 * TP, unroll=4)(_zero)
    pltpu.make_async_copy(el0_hbm.at[pl.ds(ebase, ew)], e0_v, sem0).wait()
    pltpu.make_async_copy(el1_hbm.at[pl.ds(ebase, ew)], e1_v, sem0).wait()
    pltpu.make_async_copy(et_hbm.at[pl.ds(ebase, ew)], et_v, sem0).wait()

    def _group(g):
        off = g * 16
        e0i = e0_v[pl.ds(off, 16)]
        e1i = e1_v[pl.ds(off, 16)]
        eti = et_v[pl.ds(off, 16)]
        s1g = plsc.load_gather(s1o_v, [e0i])
        s2g = plsc.load_gather(s2o_v, [e1i])
        seg = plsc.load_gather(ser_v, [eti])
        w = _lrelu_exp(s1g + s2g + seg)
        e0m = e0i * 64
        e1m = e1i * 80
        etm = eti * 64
        for j in range(16):
            fj = jnp.full((16,), j, i32)
            tB = _take(e0m, fj) + lane
            rB = _take(etm, fj) + lane
            aB = _take(e1m, fj) + lane
            wb = _take(w, fj)
            for k in range(4):
                v = (plsc.load_gather(t1o_v, [tB + k * 16])
                     + plsc.load_gather(r_v, [rB + k * 16]))
                plsc.addupdate_scatter(acc, [aB + k * 16], wb * v)
            plsc.addupdate_scatter(acc, [aB + 64], wb * lane0f)

    plsc.parallel_loop(0, ew // 16, unroll=1)(_group)

    pltpu.sync_copy(acc, out_hbm.at[pl.ds(wid * (5 * TP * 16), 5 * TP * 16)])


def _fin_body(accE2_ref, T2o_ref, x2_ref):
    a = jnp.sum(accE2_ref[...], axis=0)
    numer = a[:, 0:64]
    rs = a[:, 64:65]
    rsafe = jnp.where(rs == 0.0, 1e-12, rs)
    x2_ref[...] = _elu(numer / rsafe + jnp.where(rs > 0.0, T2o_ref[...], 0.0))


def kernel(Corpus_, batch_inputs, entity_embeddings, relation_embed, type_embed,
           edge_list, edge_type, edge_embed,
           a_h0, a2_h0, went_h0, a_h1, a2_h1, went_h1,
           a_out, a2_out, went_out, W):
    f32 = jnp.float32
    E = edge_list.shape[1]
    nch = E // (NW * CHUNK)

    entP = entity_embeddings[:TP]
    typeP = jnp.pad(type_embed, ((0, TP - NT), (0, 0)))
    relP = jnp.pad(relation_embed, ((0, RP - NR), (0, 0)))

    A1cat = jnp.concatenate([a_h0.T[0:64], a_h1.T[0:64]], axis=1)
    A2cat = jnp.concatenate([a_h0.T[64:128], a_h1.T[64:128]], axis=1)
    ArT = jnp.stack([a_h0.T[128:160], a_h1.T[128:160]])
    a2R = jnp.stack([a2_h0, a2_h1])
    wcat = jnp.concatenate([went_h0, went_h1], axis=1)
    A1o = a_out[:, 0:64]
    A2o = a_out[:, 64:128]
    Aro = a_out[:, 128:192]
    a2oR = jnp.pad(a2_out.reshape(1, 64), ((0, 7), (0, 0)))

    full = lambda shp: pl.BlockSpec(shp, lambda i: (0,) * len(shp))

    pes, T1cat, S1, S2, T2cat = pl.pallas_call(
        _prep_body,
        grid=(E // PB,),
        in_specs=[pl.BlockSpec((PB // 4, 128), lambda i: (i, 0)),
                  full((TP, 64)), full((TP, 64)), full((64, 64)),
                  full((64, 64)), full((2, 32, 32)), full((2, 32))],
        out_specs=[pl.BlockSpec((PB // 4, 512), lambda i: (i, 0)),
                   full((TP, 64)), full((8, TP)), full((8, TP)), full((TP, 64))],
        out_shape=[jax.ShapeDtypeStruct((E // 4, 512), f32),
                   jax.ShapeDtypeStruct((TP, 64), f32),
                   jax.ShapeDtypeStruct((8, TP), f32),
                   jax.ShapeDtypeStruct((8, TP), f32),
                   jax.ShapeDtypeStruct((TP, 64), f32)],
        scratch_shapes=[pltpu.VMEM((128, 512), f32)],
    )(edge_embed.reshape(E // 4, 128), entP, typeP, A1cat, A2cat, ArT, a2R)

    mesh = plsc.VectorSubcoreMesh(core_axis_name="c", subcore_axis_name="s")
    e1sc = pl.kernel(
        functools.partial(_e1sc_body, nch=nch),
        out_type=jax.ShapeDtypeStruct((NW * 2 * 3 * TP * 16,), f32),
        mesh=mesh,
        compiler_params=pltpu.CompilerParams(needs_layout_passes=False),
        scratch_types=[
            pltpu.VMEM((TP * 64,), f32),
            pltpu.VMEM((TP,), f32), pltpu.VMEM((TP,), f32),
            pltpu.VMEM((TP,), f32), pltpu.VMEM((TP,), f32),
            pltpu.VMEM((3 * TP * 16,), f32),
            pltpu.VMEM((3 * TP * 16,), f32),
            pltpu.VMEM((CHUNK * 128,), f32),
            pltpu.VMEM((CHUNK * 128,), f32),
            pltpu.VMEM((10000,), jnp.int32), pltpu.VMEM((10000,), jnp.int32),
            pltpu.SemaphoreType.DMA,
            pltpu.SemaphoreType.DMA,
        ],
    )
    el0 = edge_list[0]
    el1 = edge_list[1]
    accE1 = e1sc(pes.reshape(-1), el0, el1, T1cat.reshape(-1),
                 S1[:2].reshape(-1), S2[:2].reshape(-1))

    accE1r = accE1.reshape(NW, 2, TP, 48)
    out1, T1o, SCo, Rtab, SEr, T2o, orel = pl.pallas_call(
        _mid_body,
        grid=(10,),
        in_specs=[pl.BlockSpec((1000, 64), lambda i: (i, 0)),
                  full((64, 64)), full((64, 64)),
                  full((NW, 2, TP, 48)), full((TP, 64)), full((RP, 32)),
                  full((32, 64)), full((64, 64)), full((64, 64)),
                  full((64, 64)), full((8, 64))],
        out_specs=[pl.BlockSpec((1000, 64), lambda i: (i, 0)),
                   full((TP, 64)), full((8, TP)), full((RP, 64)),
                   full((8, RP)), full((TP, 64)), full((NR, 64))],
        out_shape=[jax.ShapeDtypeStruct((NE_ENT, 64), f32),
                   jax.ShapeDtypeStruct((TP, 64), f32),
                   jax.ShapeDtypeStruct((8, TP), f32),
                   jax.ShapeDtypeStruct((RP, 64), f32),
                   jax.ShapeDtypeStruct((8, RP), f32),
                   jax.ShapeDtypeStruct((TP, 64), f32),
                   jax.ShapeDtypeStruct((NR, 64), f32)],
    )(entity_embeddings, wcat, went_out, accE1r, T2cat, relP, W,
      A1o, A2o, Aro, a2oR)

    e2sc = pl.kernel(
        functools.partial(_e2sc_body, nch=nch),
        out_type=jax.ShapeDtypeStruct((NW * 5 * TP * 16,), f32),
        mesh=mesh,
        compiler_params=pltpu.CompilerParams(needs_layout_passes=False),
        scratch_types=[
            pltpu.VMEM((TP * 64,), f32),
            pltpu.VMEM((RP * 64,), f32),
            pltpu.VMEM((TP,), f32), pltpu.VMEM((TP,), f32),
            pltpu.VMEM((RP,), f32),
            pltpu.VMEM((5 * TP * 16,), f32),
            pltpu.VMEM((10000,), jnp.int32), pltpu.VMEM((10000,), jnp.int32),
            pltpu.VMEM((10000,), jnp.int32),
            pltpu.SemaphoreType.DMA,
        ],
    )
    accE2 = e2sc(el0, el1, edge_type, T1o.reshape(-1), Rtab.reshape(-1),
                 SCo[:2].reshape(-1), SEr[:1].reshape(-1))

    x2 = pl.pallas_call(
        _fin_body,
        grid=(1,),
        in_specs=[full((NW, TP, 80)), full((TP, 64))],
        out_specs=full((TP, 64)),
        out_shape=jax.ShapeDtypeStruct((TP, 64), f32),
    )(accE2.reshape(NW, TP, 80), T2o)

    return out1, x2[:NT], orel


# speedup vs baseline: 10.5067x; 1.1913x over previous
import functools

import jax
import jax.numpy as jnp
from jax import lax
from jax.experimental import pallas as pl
from jax.experimental.pallas import tpu as pltpu
from jax.experimental.pallas import tpu_sc as plsc

NT = 500
TP = 512
NR = 200
RP = 256
NE_ENT = 10000
PB = 3200
NW = 32
CHUNK = 80


def _elu(x):
    return jnp.where(x > 0, x, jnp.exp(x) - 1.0)


def _lrelu_exp(s):
    return jnp.exp(-jnp.where(s > 0, s, 0.2 * s))


def _take(x, idx):
    return x.at[idx].get(mode="promise_in_bounds", unique_indices=False)


def _prep_body(ee_ref, entP_ref, typeP_ref, A1cat_ref, A2cat_ref, ArT_ref,
               a2R_ref, pes_ref, T1cat_ref, S1_ref, S2_ref, T2cat_ref, W80_s):
    i = pl.program_id(0)

    @pl.when(i == 0)
    def _tables():
        T1cat = jnp.dot(entP_ref[...], A1cat_ref[...], preferred_element_type=jnp.float32)
        T2cat = jnp.dot(typeP_ref[...], A2cat_ref[...], preferred_element_type=jnp.float32)
        T1cat_ref[...] = T1cat
        T2cat_ref[...] = T2cat
        s1r, s2r, cecols = [], [], []
        for h in range(2):
            a2row = a2R_ref[h:h + 1, :]
            s1r.append(jax.lax.dot_general(
                a2row, T1cat[:, 32 * h:32 * h + 32],
                (((1,), (1,)), ((), ())), preferred_element_type=jnp.float32))
            s2r.append(jax.lax.dot_general(
                a2row, T2cat[:, 32 * h:32 * h + 32],
                (((1,), (1,)), ((), ())), preferred_element_type=jnp.float32))
            cecols.append(jax.lax.dot_general(
                ArT_ref[h], a2row,
                (((1,), (1,)), ((), ())), preferred_element_type=jnp.float32))
        S1_ref[...] = jnp.concatenate(s1r + [jnp.zeros((6, TP), jnp.float32)], axis=0)
        S2_ref[...] = jnp.concatenate(s2r + [jnp.zeros((6, TP), jnp.float32)], axis=0)
        W80_s[...] = jnp.concatenate(
            [ArT_ref[0], ArT_ref[1], cecols[0], cecols[1],
             jnp.zeros((32, 62), jnp.float32)], axis=1)

    pes_ref[...] = jnp.dot(ee_ref[...], W80_s[...], preferred_element_type=jnp.float32)


def _e1sc_body(pes_hbm, el0_hbm, el1_hbm, t1_hbm, s1_hbm, s2_hbm, out_hbm,
               t1_v, s1a, s1b, s2a, s2b, acc0, acc1, pes_a, pes_b,
               e0_v, e1_v, sem0, sem1, nch):
    i32 = jnp.int32
    wid = lax.axis_index("s") * 2 + lax.axis_index("c")
    ebase = wid * (nch * CHUNK)
    lane = lax.iota(i32, 16)
    lane0f = jnp.where(lane == 0, 1.0, 0.0).astype(jnp.float32)

    ew = nch * CHUNK
    pltpu.make_async_copy(el0_hbm.at[pl.ds(ebase, ew)], e0_v, sem0).start()
    pltpu.make_async_copy(el1_hbm.at[pl.ds(ebase, ew)], e1_v, sem0).start()
    pltpu.sync_copy(t1_hbm, t1_v)
    pltpu.sync_copy(s1_hbm.at[pl.ds(0, TP)], s1a)
    pltpu.sync_copy(s1_hbm.at[pl.ds(TP, TP)], s1b)
    pltpu.sync_copy(s2_hbm.at[pl.ds(0, TP)], s2a)
    pltpu.sync_copy(s2_hbm.at[pl.ds(TP, TP)], s2b)

    def _zero(r):
        acc0[pl.ds(r * 16, 16)] = jnp.zeros((16,), jnp.float32)
        acc1[pl.ds(r * 16, 16)] = jnp.zeros((16,), jnp.float32)
    plsc.parallel_loop(0, 3 * TP, unroll=4)(_zero)
    pltpu.make_async_copy(el0_hbm.at[pl.ds(ebase, ew)], e0_v, sem0).wait()
    pltpu.make_async_copy(el1_hbm.at[pl.ds(ebase, ew)], e1_v, sem0).wait()

    bufs = ((pes_a, sem0), (pes_b, sem1))

    def _copies(ci, b):
        base = ebase + ci * CHUNK
        pes_v, sem = bufs[b]
        return (
            pltpu.make_async_copy(pes_hbm.at[pl.ds(base * 128, CHUNK * 128)], pes_v, sem),
        )

    def _fire(ci, b):
        for c in _copies(ci, b):
            c.start()

    def _wait(ci, b):
        for c in _copies(ci, b):
            c.wait()

    def _process(ci, b):
        pes_v, _ = bufs[b]
        coff = ci * CHUNK

        def _group(g, _):
            off = g * 16
            e0i = e0_v[pl.ds(coff + off, 16)]
            e1i = e1_v[pl.ds(coff + off, 16)]
            s1g0 = plsc.load_gather(s1a, [e0i])
            s1g1 = plsc.load_gather(s1b, [e0i])
            s2g0 = plsc.load_gather(s2a, [e1i])
            s2g1 = plsc.load_gather(s2b, [e1i])
            pbase = (lane + off) * 128
            se0 = plsc.load_gather(pes_v, [pbase + 64])
            se1 = plsc.load_gather(pes_v, [pbase + 65])
            w0 = _lrelu_exp(s1g0 + s2g0 + se0)
            w1 = _lrelu_exp(s1g1 + s2g1 + se1)
            e0m = e0i * 64
            e1m = e1i * 48
            for j in range(16):
                fj = jnp.full((16,), j, i32)
                tB = _take(e0m, fj) + lane
                aB = _take(e1m, fj) + lane
                w0b = _take(w0, fj)
                w1b = _take(w1, fj)
                pr = (off + j) * 128
                for h, (wb, acc) in enumerate(((w0b, acc0), (w1b, acc1))):
                    t0 = plsc.load_gather(t1_v, [tB + 32 * h])
                    t1r = plsc.load_gather(t1_v, [tB + (32 * h + 16)])
                    p0 = pes_v[pl.ds(pr + 32 * h, 16)]
                    p1 = pes_v[pl.ds(pr + 32 * h + 16, 16)]
                    plsc.addupdate_scatter(acc, [aB], wb * (t0 + p0))
                    plsc.addupdate_scatter(acc, [aB + 16], wb * (t1r + p1))
                    plsc.addupdate_scatter(acc, [aB + 32], wb * lane0f)
            return 0
        plsc.parallel_loop(0, CHUNK // 16, unroll=1)(
            lambda g: (_group(g, 0), None)[1])

    _fire(0, 0)

    def _pair(p, _):
        ci0 = 2 * p
        _fire(ci0 + 1, 1)
        _wait(ci0, 0)
        _process(ci0, 0)
        _fire(ci0 + 2, 0)
        _wait(ci0 + 1, 1)
        _process(ci0 + 1, 1)
        return 0
    lax.fori_loop(0, (nch - 1) // 2, _pair, 0)
    _wait(nch - 1, 0)
    _process(nch - 1, 0)

    pltpu.sync_copy(acc0, out_hbm.at[pl.ds((wid * 2) * (3 * TP * 16), 3 * TP * 16)])
    pltpu.sync_copy(acc1, out_hbm.at[pl.ds((wid * 2 + 1) * (3 * TP * 16), 3 * TP * 16)])


def _mid_body(ent_ref, wcat_ref, wout_ref, accE1_ref, T2cat_ref, relP_ref,
              W_ref, A1o_ref, A2o_ref, Aro_ref, a2o_ref,
              out1_ref, T1o_ref, SCo_ref, Rtab_ref, SEr_ref, T2o_ref, orel_ref):
    i = pl.program_id(0)
    h1 = jnp.dot(ent_ref[...], wcat_ref[...], preferred_element_type=jnp.float32)
    x1c = _elu(h1)
    out1_ref[...] = _elu(jnp.dot(x1c, wout_ref[...], preferred_element_type=jnp.float32))

    @pl.when(i == 0)
    def _tables():
        acc = jnp.sum(accE1_ref[...], axis=0)
        h2s = []
        for h in range(2):
            numer = acc[h, :, 0:32]
            rs = acc[h, :, 32:33]
            rsafe = jnp.where(rs == 0.0, 1e-12, rs)
            h2 = numer / rsafe + jnp.where(
                rs > 0.0, T2cat_ref[:, 32 * h:32 * h + 32], 0.0)
            h2s.append(_elu(h2))
        x2c = jnp.concatenate(h2s, axis=1)
        a2o = a2o_ref[0:1, :]
        x1c512 = x1c[:TP, :]
        T1o = jax.lax.dot_general(x1c512, A1o_ref[...],
                                  (((1,), (1,)), ((), ())), preferred_element_type=jnp.float32)
        T1o_ref[...] = T1o
        s1o = jax.lax.dot_general(a2o, T1o, (((1,), (1,)), ((), ())),
                                  preferred_element_type=jnp.float32)
        T2o = jax.lax.dot_general(x2c, A2o_ref[...],
                                  (((1,), (1,)), ((), ())), preferred_element_type=jnp.float32)
        T2o_ref[...] = T2o
        s2o = jax.lax.dot_general(a2o, T2o, (((1,), (1,)), ((), ())),
                                  preferred_element_type=jnp.float32)
        SCo_ref[...] = jnp.concatenate(
            [s1o, s2o, jnp.zeros((6, TP), jnp.float32)], axis=0)
        orelP = jnp.dot(relP_ref[...], W_ref[...], preferred_element_type=jnp.float32)
        orel_ref[...] = orelP[:NR, :]
        Rtab = jax.lax.dot_general(orelP, Aro_ref[...],
                                   (((1,), (1,)), ((), ())), preferred_element_type=jnp.float32)
        Rtab_ref[...] = Rtab
        ser = jax.lax.dot_general(a2o, Rtab, (((1,), (1,)), ((), ())),
                                  preferred_element_type=jnp.float32)
        SEr_ref[...] = jnp.concatenate(
            [ser, jnp.zeros((7, RP), jnp.float32)], axis=0)


def _e2sc_body(el0_hbm, el1_hbm, et_hbm, t1o_hbm, r_hbm, sco_hbm, ser_hbm,
               out_hbm, t1o_v, r_v, s1o_v, s2o_v, ser_v, acc,
               e0_v, e1_v, et_v, sem0, nch):
    i32 = jnp.int32
    wid = lax.axis_index("s") * 2 + lax.axis_index("c")
    ebase = wid * (nch * CHUNK)
    ew = nch * CHUNK
    lane = lax.iota(i32, 16)
    lane0f = jnp.where(lane == 0, 1.0, 0.0).astype(jnp.float32)

    pltpu.make_async_copy(el0_hbm.at[pl.ds(ebase, ew)], e0_v, sem0).start()
    pltpu.make_async_copy(el1_hbm.at[pl.ds(ebase, ew)], e1_v, sem0).start()
    pltpu.make_async_copy(et_hbm.at[pl.ds(ebase, ew)], et_v, sem0).start()
    pltpu.sync_copy(t1o_hbm, t1o_v)
    pltpu.sync_copy(r_hbm, r_v)
    pltpu.sync_copy(sco_hbm.at[pl.ds(0, TP)], s1o_v)
    pltpu.sync_copy(sco_hbm.at[pl.ds(TP, TP)], s2o_v)
    pltpu.sync_copy(ser_hbm.at[pl.ds(0, RP)], ser_v)

    def _zero(r):
        acc[pl.ds(r * 16, 16)] = jnp.zeros((16,), jnp.float32)
    plsc.parallel_loop(0, 5 * TP, unroll=4)(_zero)
    pltpu.make_async_copy(el0_hbm.at[pl.ds(ebase, ew)], e0_v, sem0).wait()
    pltpu.make_async_copy(el1_hbm.at[pl.ds(ebase, ew)], e1_v, sem0).wait()
    pltpu.make_async_copy(et_hbm.at[pl.ds(ebase, ew)], et_v, sem0).wait()

    def _group(g):
        off = g * 16
        e0i = e0_v[pl.ds(off, 16)]
        e1i = e1_v[pl.ds(off, 16)]
        eti = et_v[pl.ds(off, 16)]
        s1g = plsc.load_gather(s1o_v, [e0i])
        s2g = plsc.load_gather(s2o_v, [e1i])
        seg = plsc.load_gather(ser_v, [eti])
        w = _lrelu_exp(s1g + s2g + seg)
        e0m = e0i * 64
        e1m = e1i * 80
        etm = eti * 64
        for j in range(16):
            fj = jnp.full((16,), j, i32)
            tB = _take(e0m, fj) + lane
            rB = _take(etm, fj) + lane
            aB = _take(e1m, fj) + lane
            wb = _take(w, fj)
            for k in range(4):
                v = (plsc.load_gather(t1o_v, [tB + k * 16])
                     + plsc.load_gather(r_v, [rB + k * 16]))
                plsc.addupdate_scatter(acc, [aB + k * 16], wb * v)
            plsc.addupdate_scatter(acc, [aB + 64], wb * lane0f)

    plsc.parallel_loop(0, ew // 16, unroll=1)(_group)

    pltpu.sync_copy(acc, out_hbm.at[pl.ds(wid * (5 * TP * 16), 5 * TP * 16)])


def _fin_body(accE2_ref, T2o_ref, x2_ref):
    a = jnp.sum(accE2_ref[...], axis=0)
    numer = a[:, 0:64]
    rs = a[:, 64:65]
    rsafe = jnp.where(rs == 0.0, 1e-12, rs)
    x2_ref[...] = _elu(numer / rsafe + jnp.where(rs > 0.0, T2o_ref[...], 0.0))


def kernel(Corpus_, batch_inputs, entity_embeddings, relation_embed, type_embed,
           edge_list, edge_type, edge_embed,
           a_h0, a2_h0, went_h0, a_h1, a2_h1, went_h1,
           a_out, a2_out, went_out, W):
    f32 = jnp.float32
    E = edge_list.shape[1]
    nch = E // (NW * CHUNK)

    entP = entity_embeddings[:TP]
    typeP = jnp.pad(type_embed, ((0, TP - NT), (0, 0)))
    relP = jnp.pad(relation_embed, ((0, RP - NR), (0, 0)))

    A1cat = jnp.concatenate([a_h0.T[0:64], a_h1.T[0:64]], axis=1)
    A2cat = jnp.concatenate([a_h0.T[64:128], a_h1.T[64:128]], axis=1)
    ArT = jnp.stack([a_h0.T[128:160], a_h1.T[128:160]])
    a2R = jnp.stack([a2_h0, a2_h1])
    wcat = jnp.concatenate([went_h0, went_h1], axis=1)
    A1o = a_out[:, 0:64]
    A2o = a_out[:, 64:128]
    Aro = a_out[:, 128:192]
    a2oR = jnp.pad(a2_out.reshape(1, 64), ((0, 7), (0, 0)))

    full = lambda shp: pl.BlockSpec(shp, lambda i: (0,) * len(shp))

    pes, T1cat, S1, S2, T2cat = pl.pallas_call(
        _prep_body,
        grid=(E // PB,),
        in_specs=[pl.BlockSpec((PB, 32), lambda i: (i, 0)),
                  full((TP, 64)), full((TP, 64)), full((64, 64)),
                  full((64, 64)), full((2, 32, 32)), full((2, 32))],
        out_specs=[pl.BlockSpec((PB, 128), lambda i: (i, 0)),
                   full((TP, 64)), full((8, TP)), full((8, TP)), full((TP, 64))],
        out_shape=[jax.ShapeDtypeStruct((E, 128), f32),
                   jax.ShapeDtypeStruct((TP, 64), f32),
                   jax.ShapeDtypeStruct((8, TP), f32),
                   jax.ShapeDtypeStruct((8, TP), f32),
                   jax.ShapeDtypeStruct((TP, 64), f32)],
        scratch_shapes=[pltpu.VMEM((32, 128), f32)],
    )(edge_embed, entP, typeP, A1cat, A2cat, ArT, a2R)

    mesh = plsc.VectorSubcoreMesh(core_axis_name="c", subcore_axis_name="s")
    e1sc = pl.kernel(
        functools.partial(_e1sc_body, nch=nch),
        out_type=jax.ShapeDtypeStruct((NW * 2 * 3 * TP * 16,), f32),
        mesh=mesh,
        compiler_params=pltpu.CompilerParams(needs_layout_passes=False),
        scratch_types=[
            pltpu.VMEM((TP * 64,), f32),
            pltpu.VMEM((TP,), f32), pltpu.VMEM((TP,), f32),
            pltpu.VMEM((TP,), f32), pltpu.VMEM((TP,), f32),
            pltpu.VMEM((3 * TP * 16,), f32),
            pltpu.VMEM((3 * TP * 16,), f32),
            pltpu.VMEM((CHUNK * 128,), f32),
            pltpu.VMEM((CHUNK * 128,), f32),
            pltpu.VMEM((10000,), jnp.int32), pltpu.VMEM((10000,), jnp.int32),
            pltpu.SemaphoreType.DMA,
            pltpu.SemaphoreType.DMA,
        ],
    )
    el0 = edge_list[0]
    el1 = edge_list[1]
    accE1 = e1sc(pes.reshape(-1), el0, el1, T1cat.reshape(-1),
                 S1[:2].reshape(-1), S2[:2].reshape(-1))

    accE1r = accE1.reshape(NW, 2, TP, 48)
    out1, T1o, SCo, Rtab, SEr, T2o, orel = pl.pallas_call(
        _mid_body,
        grid=(10,),
        in_specs=[pl.BlockSpec((1000, 64), lambda i: (i, 0)),
                  full((64, 64)), full((64, 64)),
                  full((NW, 2, TP, 48)), full((TP, 64)), full((RP, 32)),
                  full((32, 64)), full((64, 64)), full((64, 64)),
                  full((64, 64)), full((8, 64))],
        out_specs=[pl.BlockSpec((1000, 64), lambda i: (i, 0)),
                   full((TP, 64)), full((8, TP)), full((RP, 64)),
                   full((8, RP)), full((TP, 64)), full((NR, 64))],
        out_shape=[jax.ShapeDtypeStruct((NE_ENT, 64), f32),
                   jax.ShapeDtypeStruct((TP, 64), f32),
                   jax.ShapeDtypeStruct((8, TP), f32),
                   jax.ShapeDtypeStruct((RP, 64), f32),
                   jax.ShapeDtypeStruct((8, RP), f32),
                   jax.ShapeDtypeStruct((TP, 64), f32),
                   jax.ShapeDtypeStruct((NR, 64), f32)],
    )(entity_embeddings, wcat, went_out, accE1r, T2cat, relP, W,
      A1o, A2o, Aro, a2oR)

    e2sc = pl.kernel(
        functools.partial(_e2sc_body, nch=nch),
        out_type=jax.ShapeDtypeStruct((NW * 5 * TP * 16,), f32),
        mesh=mesh,
        compiler_params=pltpu.CompilerParams(needs_layout_passes=False),
        scratch_types=[
            pltpu.VMEM((TP * 64,), f32),
            pltpu.VMEM((RP * 64,), f32),
            pltpu.VMEM((TP,), f32), pltpu.VMEM((TP,), f32),
            pltpu.VMEM((RP,), f32),
            pltpu.VMEM((5 * TP * 16,), f32),
            pltpu.VMEM((10000,), jnp.int32), pltpu.VMEM((10000,), jnp.int32),
            pltpu.VMEM((10000,), jnp.int32),
            pltpu.SemaphoreType.DMA,
        ],
    )
    accE2 = e2sc(el0, el1, edge_type, T1o.reshape(-1), Rtab.reshape(-1),
                 SCo[:2].reshape(-1), SEr[:1].reshape(-1))

    x2 = pl.pallas_call(
        _fin_body,
        grid=(1,),
        in_specs=[full((NW, TP, 80)), full((TP, 64))],
        out_specs=full((TP, 64)),
        out_shape=jax.ShapeDtypeStruct((TP, 64), f32),
    )(accE2.reshape(NW, TP, 80), T2o)

    return out1, x2[:NT], orel
